# Initial kernel scaffold; baseline (speedup 1.0000x reference)
import jax
import jax.numpy as jnp
from jax.experimental import pallas as pl


def kernel(hidden_states, gate_weight, gate_up_proj, down_proj):
    raise NotImplementedError("write your pallas kernel here")


# speedup vs baseline: 1.2784x; 1.2784x over previous
import functools

import jax
import jax.numpy as jnp
from jax.experimental import pallas as pl

NUM_EXPERTS = 8
HIDDEN = 1024
INTER = 768
TOP_K = 2


def _moe_body(x_ref, gwt_ref, gup_ref, dnt_ref, out_ref):
    e = pl.program_id(0)
    x = x_ref[...]

    logits = jax.lax.dot_general(
        x, gwt_ref[...], (((1,), (0,)), ((), ())),
        precision=jax.lax.Precision.DEFAULT,
        preferred_element_type=jnp.float32)
    m = jnp.max(logits, axis=-1, keepdims=True)
    p = jnp.exp(logits - m)
    w = p / jnp.sum(p, axis=-1, keepdims=True)

    iota = jax.lax.broadcasted_iota(jnp.int32, w.shape, 1)
    w1 = jnp.max(w, axis=-1, keepdims=True)
    i1 = jnp.min(jnp.where(w == w1, iota, NUM_EXPERTS), axis=-1, keepdims=True)
    wm = jnp.where(iota == i1, -jnp.inf, w)
    w2 = jnp.max(wm, axis=-1, keepdims=True)
    i2 = jnp.min(jnp.where(wm == w2, iota, NUM_EXPERTS), axis=-1, keepdims=True)
    ce = jnp.where(i1 == e, w1, 0.0) + jnp.where(i2 == e, w2, 0.0)

    xb = x.astype(jnp.bfloat16)
    gu = jax.lax.dot_general(
        xb, gup_ref[0], (((1,), (0,)), ((), ())),
        preferred_element_type=jnp.float32)
    g = gu[:, :INTER]
    u = gu[:, INTER:]
    h = (g * (1.0 / (1.0 + jnp.exp(-g))) * u).astype(jnp.bfloat16)
    oe = jax.lax.dot_general(
        h, dnt_ref[0], (((1,), (0,)), ((), ())),
        preferred_element_type=jnp.float32)

    contrib = ce * oe

    @pl.when(e == 0)
    def _():
        out_ref[...] = contrib

    @pl.when(e > 0)
    def _():
        out_ref[...] += contrib


@functools.partial(jax.jit, static_argnames=())
def kernel(hidden_states, gate_weight, gate_up_proj, down_proj):
    Bb, Ss, H = hidden_states.shape
    T = Bb * Ss
    x = hidden_states.reshape(T, H)
    gwt = gate_weight.T
    gup_t = gate_up_proj.transpose(0, 2, 1).astype(jnp.bfloat16)
    dnt = down_proj.transpose(0, 2, 1).astype(jnp.bfloat16)

    out = pl.pallas_call(
        _moe_body,
        grid=(NUM_EXPERTS,),
        in_specs=[
            pl.BlockSpec((T, H), lambda e: (0, 0)),
            pl.BlockSpec((H, NUM_EXPERTS), lambda e: (0, 0)),
            pl.BlockSpec((1, H, 2 * INTER), lambda e: (e, 0, 0)),
            pl.BlockSpec((1, INTER, H), lambda e: (e, 0, 0)),
        ],
        out_specs=pl.BlockSpec((T, H), lambda e: (0, 0)),
        out_shape=jax.ShapeDtypeStruct((T, H), jnp.float32),
    )(x, gwt, gup_t, dnt)
    return out.reshape(Bb, Ss, H)


# speedup vs baseline: 1.3988x; 1.0942x over previous
import functools

import jax
import jax.numpy as jnp
from jax.experimental import pallas as pl
from jax.experimental.pallas import tpu as pltpu

NUM_EXPERTS = 8
HIDDEN = 1024
INTER = 768
TOP_K = 2


def _router_comb(x, gwt):
    logits = jax.lax.dot_general(
        x, gwt, (((1,), (0,)), ((), ())),
        precision=jax.lax.Precision.DEFAULT,
        preferred_element_type=jnp.float32)
    m = jnp.max(logits, axis=-1, keepdims=True)
    p = jnp.exp(logits - m)
    w = p / jnp.sum(p, axis=-1, keepdims=True)

    iota = jax.lax.broadcasted_iota(jnp.int32, w.shape, 1)
    w1 = jnp.max(w, axis=-1, keepdims=True)
    i1 = jnp.min(jnp.where(w == w1, iota, NUM_EXPERTS), axis=-1, keepdims=True)
    wm = jnp.where(iota == i1, -jnp.inf, w)
    w2 = jnp.max(wm, axis=-1, keepdims=True)
    i2 = jnp.min(jnp.where(wm == w2, iota, NUM_EXPERTS), axis=-1, keepdims=True)
    return jnp.where(iota == i1, w1, 0.0) + jnp.where(iota == i2, w2, 0.0)


def _moe_body(x_ref, gwt_ref, gup_ref, dnt_ref, out_ref, comb_ref):
    e = pl.program_id(0)
    x = x_ref[...]

    @pl.when(e == 0)
    def _():
        comb_ref[...] = _router_comb(x, gwt_ref[...])

    onehot = (jax.lax.broadcasted_iota(jnp.int32, (NUM_EXPERTS, 1), 0)
              == e).astype(jnp.float32)
    ce = jax.lax.dot_general(
        comb_ref[...], onehot, (((1,), (0,)), ((), ())),
        preferred_element_type=jnp.float32)

    xb = x.astype(jnp.bfloat16)
    gu = jax.lax.dot_general(
        xb, gup_ref[0], (((1,), (0,)), ((), ())),
        preferred_element_type=jnp.float32)
    g = gu[:, :INTER]
    u = gu[:, INTER:]
    h = (g * (1.0 / (1.0 + jnp.exp(-g))) * u).astype(jnp.bfloat16)
    oe = jax.lax.dot_general(
        h, dnt_ref[0], (((1,), (0,)), ((), ())),
        preferred_element_type=jnp.float32)

    contrib = ce * oe

    @pl.when(e == 0)
    def _():
        out_ref[...] = contrib

    @pl.when(e > 0)
    def _():
        out_ref[...] += contrib


@functools.partial(jax.jit, static_argnames=())
def kernel(hidden_states, gate_weight, gate_up_proj, down_proj):
    Bb, Ss, H = hidden_states.shape
    T = Bb * Ss
    x = hidden_states.reshape(T, H)
    gwt = gate_weight.T
    gup_t = gate_up_proj.transpose(0, 2, 1).astype(jnp.bfloat16)
    dnt = down_proj.transpose(0, 2, 1).astype(jnp.bfloat16)

    out = pl.pallas_call(
        _moe_body,
        grid=(NUM_EXPERTS,),
        in_specs=[
            pl.BlockSpec((T, H), lambda e: (0, 0)),
            pl.BlockSpec((H, NUM_EXPERTS), lambda e: (0, 0)),
            pl.BlockSpec((1, H, 2 * INTER), lambda e: (e, 0, 0)),
            pl.BlockSpec((1, INTER, H), lambda e: (e, 0, 0)),
        ],
        out_specs=pl.BlockSpec((T, H), lambda e: (0, 0)),
        out_shape=jax.ShapeDtypeStruct((T, H), jnp.float32),
        scratch_shapes=[pltpu.VMEM((T, NUM_EXPERTS), jnp.float32)],
    )(x, gwt, gup_t, dnt)
    return out.reshape(Bb, Ss, H)
